# Initial kernel scaffold; baseline (speedup 1.0000x reference)
import jax
import jax.numpy as jnp
from jax.experimental import pallas as pl


def kernel(feature_matrix, edge_index, W):
    raise NotImplementedError("write your pallas kernel here")


# speedup vs baseline: 3.5464x; 3.5464x over previous
import jax
import jax.numpy as jnp
from jax import lax
from jax.experimental import pallas as pl
from jax.experimental.pallas import tpu as pltpu
from jax.experimental.pallas import tpu_sc as plsc

N_NODES = 10000
N_EDGES = 320000
D = 128

CHUNK = 128
NC = 2
NS = 16
N_TILES = NC * NS
CPT = 80
HALF = CPT // 2
N_CHUNKS = CPT * N_TILES
PAD_DST = N_NODES
N_ACC = 10112
ROWS_PER_SUB = N_ACC // NS

_mesh = plsc.VectorSubcoreMesh(core_axis_name="core", subcore_axis_name="subcore")


def _sc_body(x_hbm, src_hbm, dst_hbm, part_hbm,
             acc, src_idx, dst_idx, rows0, rows1,
             sem_s, sem_d, sem0, sem1):
    c = lax.axis_index("core")
    s = lax.axis_index("subcore")
    t = c * NS + s

    cp_s = pltpu.async_copy(src_hbm.at[pl.ds(t * CPT, HALF)], src_idx, sem_s)
    cp_d = pltpu.async_copy(dst_hbm.at[pl.ds(t * CPT, HALF)], dst_idx, sem_d)

    zero16 = jnp.zeros((16,), jnp.float32)

    @pl.loop(0, CHUNK)
    def _zero_rows(i):
        for j in range(0, D, 16):
            rows0[i, pl.ds(j, 16)] = zero16

    for k in range(ROWS_PER_SUB // CHUNK):
        pltpu.sync_copy(rows0, acc.at[pl.ds(s * ROWS_PER_SUB + k * CHUNK, CHUNK)])
    _tail = ROWS_PER_SUB % CHUNK
    if _tail:
        pltpu.sync_copy(
            rows0.at[pl.ds(0, _tail)],
            acc.at[pl.ds(s * ROWS_PER_SUB + ROWS_PER_SUB - _tail, _tail)])

    plsc.subcore_barrier()
    cp_s.wait()
    cp_d.wait()

    for h in range(CPT // HALF):
        if h > 0:
            pltpu.sync_copy(src_hbm.at[pl.ds(t * CPT + h * HALF, HALF)], src_idx)
            pltpu.sync_copy(dst_hbm.at[pl.ds(t * CPT + h * HALF, HALF)], dst_idx)

        pltpu.async_copy(x_hbm.at[src_idx.at[0]], rows0, sem0)
        pltpu.async_copy(x_hbm.at[src_idx.at[1]], rows1, sem1)

        @pl.loop(0, HALF, step=2)
        def _edges(i):
            pltpu.make_async_copy(x_hbm.at[src_idx.at[i]], rows0, sem0).wait()

            @pl.when(i + 2 < HALF)
            def _():
                pltpu.async_copy(x_hbm.at[src_idx.at[i + 2]], rows0, sem0)

            pltpu.sync_copy(rows0, acc.at[dst_idx.at[i]], add=True)

            pltpu.make_async_copy(x_hbm.at[src_idx.at[i + 1]], rows1, sem1).wait()

            @pl.when(i + 3 < HALF)
            def _():
                pltpu.async_copy(x_hbm.at[src_idx.at[i + 3]], rows1, sem1)

            pltpu.sync_copy(rows1, acc.at[dst_idx.at[i + 1]], add=True)

    plsc.subcore_barrier()

    pltpu.sync_copy(acc.at[pl.ds(s * ROWS_PER_SUB, ROWS_PER_SUB)],
                    part_hbm.at[c].at[pl.ds(s * ROWS_PER_SUB, ROWS_PER_SUB)])


_sc_aggregate = pl.kernel(
    _sc_body,
    out_type=jax.ShapeDtypeStruct((NC, N_ACC, D), jnp.float32),
    mesh=_mesh,
    scratch_types=[
        pltpu.VMEM_SHARED((N_ACC, D), jnp.float32),
        pltpu.VMEM((HALF, CHUNK), jnp.int32),
        pltpu.VMEM((HALF, CHUNK), jnp.int32),
        pltpu.VMEM((CHUNK, D), jnp.float32),
        pltpu.VMEM((CHUNK, D), jnp.float32),
        pltpu.SemaphoreType.DMA,
        pltpu.SemaphoreType.DMA,
        pltpu.SemaphoreType.DMA,
        pltpu.SemaphoreType.DMA,
    ],
)


def _combine_body(p0, p1, w, o):
    o[...] = (p0[0] + p1[0]) * w[...]


def _combine(part, w2d):
    blk = 1000
    return pl.pallas_call(
        _combine_body,
        out_shape=jax.ShapeDtypeStruct((N_NODES, D), jnp.float32),
        grid=(N_NODES // blk,),
        in_specs=[
            pl.BlockSpec((1, blk, D), lambda i: (0, i, 0)),
            pl.BlockSpec((1, blk, D), lambda i: (1, i, 0)),
            pl.BlockSpec((1, D), lambda i: (0, 0)),
        ],
        out_specs=pl.BlockSpec((blk, D), lambda i: (i, 0)),
    )(part, part, w2d)


def kernel(feature_matrix, edge_index, W):
    n_pad = N_CHUNKS * CHUNK - N_EDGES
    src = jnp.concatenate(
        [edge_index[0], jnp.zeros((n_pad,), jnp.int32)]).reshape(N_CHUNKS, CHUNK)
    dst = jnp.concatenate(
        [edge_index[1], jnp.full((n_pad,), PAD_DST, jnp.int32)]).reshape(N_CHUNKS, CHUNK)
    part = _sc_aggregate(feature_matrix, src, dst)
    return _combine(part, W.reshape(1, D))
